# Initial kernel scaffold; baseline (speedup 1.0000x reference)
import functools

import jax
import jax.numpy as jnp
from jax import lax
from jax.experimental import pallas as pl
from jax.experimental.pallas import tpu as pltpu
from jax.experimental.pallas import tpu_sc as plsc

VOCAB = 1_000_000
EMB = 16
ROWS = 16384
COLS = 200
B_TOTAL = ROWS * COLS

_info = plsc.get_sparse_core_info()
NUM_CORES = _info.num_cores
NUM_SUBCORES = _info.num_subcores
NW = NUM_CORES * NUM_SUBCORES
B_PER_W = B_TOTAL // NW
CHUNK = 2048
N_CHUNKS = B_PER_W // CHUNK


def _make_kernel():
    mesh = plsc.VectorSubcoreMesh(core_axis_name="c", subcore_axis_name="s")

    @functools.partial(
        pl.kernel,
        mesh=mesh,
        out_type=jax.ShapeDtypeStruct((B_TOTAL, EMB), jnp.float32),
        scratch_types=[
            pltpu.VMEM((CHUNK,), jnp.int32),
            pltpu.VMEM((CHUNK, EMB), jnp.float32),
            pltpu.SemaphoreType.DMA,
        ],
    )
    def emb_kernel(x_hbm, table_hbm, out_hbm, idx_v, rows_v, sem):
        wid = lax.axis_index("s") * NUM_CORES + lax.axis_index("c")
        base = wid * B_PER_W

        def body(g, _):
            off = base + g * CHUNK
            pltpu.sync_copy(x_hbm.at[pl.ds(off, CHUNK)], idx_v)
            pltpu.async_copy(table_hbm.at[idx_v], rows_v, sem).wait()
            pltpu.sync_copy(rows_v, out_hbm.at[pl.ds(off, CHUNK)])
            return 0

        lax.fori_loop(0, N_CHUNKS, body, 0)

    return emb_kernel


_emb_kernel = _make_kernel()


def kernel(x, table):
    flat = _emb_kernel(x.reshape(B_TOTAL), table)
    return flat.reshape(ROWS, COLS, EMB)


# speedup vs baseline: 2.4854x; 2.4854x over previous
import functools

import jax
import jax.numpy as jnp
from jax import lax
from jax.experimental import pallas as pl
from jax.experimental.pallas import tpu as pltpu
from jax.experimental.pallas import tpu_sc as plsc

VOCAB = 1_000_000
EMB = 16
ROWS = 16384
COLS = 200
B_TOTAL = ROWS * COLS

_info = plsc.get_sparse_core_info()
NUM_CORES = _info.num_cores
NUM_SUBCORES = _info.num_subcores
NW = NUM_CORES * NUM_SUBCORES
B_PER_W = B_TOTAL // NW
CHUNK = 2048
N_CHUNKS = B_PER_W // CHUNK


def _make_kernel():
    mesh = plsc.VectorSubcoreMesh(core_axis_name="c", subcore_axis_name="s")

    @functools.partial(
        pl.kernel,
        mesh=mesh,
        out_type=jax.ShapeDtypeStruct((B_TOTAL, EMB), jnp.float32),
        scratch_types=[
            pltpu.VMEM((CHUNK,), jnp.int32),
            pltpu.VMEM((CHUNK, EMB), jnp.float32),
            pltpu.SemaphoreType.DMA,
        ],
        compiler_params=pltpu.CompilerParams(use_tc_tiling_on_sc=False),
    )
    def emb_kernel(x_hbm, table_hbm, out_hbm, idx_v, rows_v, sem):
        wid = lax.axis_index("s") * NUM_CORES + lax.axis_index("c")
        base = wid * B_PER_W

        def body(g, _):
            off = base + g * CHUNK
            pltpu.sync_copy(x_hbm.at[pl.ds(off, CHUNK)], idx_v)
            pltpu.async_copy(table_hbm.at[idx_v], rows_v, sem).wait()
            pltpu.sync_copy(rows_v, out_hbm.at[pl.ds(off, CHUNK)])
            return 0

        lax.fori_loop(0, N_CHUNKS, body, 0)

    return emb_kernel


_emb_kernel = _make_kernel()


def kernel(x, table):
    flat = _emb_kernel(x.reshape(B_TOTAL), table)
    return flat.reshape(ROWS, COLS, EMB)


# speedup vs baseline: 2.5685x; 1.0334x over previous
import functools

import jax
import jax.numpy as jnp
from jax import lax
from jax.experimental import pallas as pl
from jax.experimental.pallas import tpu as pltpu
from jax.experimental.pallas import tpu_sc as plsc

VOCAB = 1_000_000
EMB = 16
ROWS = 16384
COLS = 200
B_TOTAL = ROWS * COLS

_info = plsc.get_sparse_core_info()
NUM_CORES = _info.num_cores
NUM_SUBCORES = _info.num_subcores
NW = NUM_CORES * NUM_SUBCORES
B_PER_W = B_TOTAL // NW
CHUNK = 2048
N_CHUNKS = B_PER_W // CHUNK


def _make_kernel():
    mesh = plsc.VectorSubcoreMesh(core_axis_name="c", subcore_axis_name="s")

    @functools.partial(
        pl.kernel,
        mesh=mesh,
        out_type=jax.ShapeDtypeStruct((B_TOTAL, EMB), jnp.float32),
        scratch_types=[
            pltpu.VMEM((2, CHUNK), jnp.int32),
            pltpu.VMEM((2, CHUNK, EMB), jnp.float32),
            pltpu.SemaphoreType.DMA,
            pltpu.SemaphoreType.DMA,
            pltpu.SemaphoreType.DMA,
            pltpu.SemaphoreType.DMA,
            pltpu.SemaphoreType.DMA,
            pltpu.SemaphoreType.DMA,
        ],
        compiler_params=pltpu.CompilerParams(use_tc_tiling_on_sc=False),
    )
    def emb_kernel(x_hbm, table_hbm, out_hbm, idx_v, rows_v,
                   ld0, ld1, g0, g1, st0, st1):
        sem_ld = (ld0, ld1)
        sem_g = (g0, g1)
        sem_st = (st0, st1)
        wid = lax.axis_index("s") * NUM_CORES + lax.axis_index("c")
        base = wid * B_PER_W

        def fire_ld(g, b):
            pltpu.async_copy(
                x_hbm.at[pl.ds(base + g * CHUNK, CHUNK)], idx_v.at[b],
                sem_ld[b])

        def wait_ld(b):
            pltpu.make_async_copy(
                x_hbm.at[pl.ds(base, CHUNK)], idx_v.at[b], sem_ld[b]).wait()

        def fire_gather(b):
            pltpu.async_copy(table_hbm.at[idx_v.at[b]], rows_v.at[b],
                             sem_g[b])

        def wait_gather(b):
            pltpu.make_async_copy(table_hbm.at[idx_v.at[b]], rows_v.at[b],
                                  sem_g[b]).wait()

        def fire_st(g, b):
            pltpu.async_copy(
                rows_v.at[b], out_hbm.at[pl.ds(base + g * CHUNK, CHUNK)],
                sem_st[b])

        def wait_st(b):
            pltpu.make_async_copy(
                rows_v.at[b], out_hbm.at[pl.ds(base, CHUNK)],
                sem_st[b]).wait()

        fire_ld(0, 0)
        fire_ld(1, 1)
        wait_ld(0)
        fire_gather(0)

        wait_ld(1)
        fire_gather(1)
        wait_gather(0)
        fire_st(0, 0)
        fire_ld(2, 0)

        wait_ld(0)
        wait_st(0)
        fire_gather(0)
        wait_gather(1)
        fire_st(1, 1)
        fire_ld(3, 1)

        def steady(g, buf):
            nb = 1 - buf
            wait_ld(nb)
            wait_st(nb)
            fire_gather(nb)
            wait_gather(buf)
            fire_st(g, buf)
            fire_ld(g + 2, buf)

        @pl.loop(2, N_CHUNKS - 2, step=2)
        def _(h):
            for b_off in range(2):
                steady(h + b_off, b_off)

        wait_ld(1)
        wait_st(1)
        fire_gather(1)
        wait_gather(0)
        fire_st(N_CHUNKS - 2, 0)

        wait_gather(1)
        fire_st(N_CHUNKS - 1, 1)

        wait_st(0)
        wait_st(1)

    return emb_kernel


_emb_kernel = _make_kernel()


def kernel(x, table):
    flat = _emb_kernel(x.reshape(B_TOTAL), table)
    return flat.reshape(ROWS, COLS, EMB)


# speedup vs baseline: 2.5699x; 1.0005x over previous
import functools

import jax
import jax.numpy as jnp
from jax import lax
from jax.experimental import pallas as pl
from jax.experimental.pallas import tpu as pltpu
from jax.experimental.pallas import tpu_sc as plsc

VOCAB = 1_000_000
EMB = 16
ROWS = 16384
COLS = 200
B_TOTAL = ROWS * COLS

_info = plsc.get_sparse_core_info()
NUM_CORES = _info.num_cores
NUM_SUBCORES = _info.num_subcores
NW = NUM_CORES * NUM_SUBCORES
B_PER_W = B_TOTAL // NW
NBUF = 4
CHUNK = 1600
N_CHUNKS = B_PER_W // CHUNK


def _make_kernel():
    mesh = plsc.VectorSubcoreMesh(core_axis_name="c", subcore_axis_name="s")

    @functools.partial(
        pl.kernel,
        mesh=mesh,
        out_type=jax.ShapeDtypeStruct((B_TOTAL, EMB), jnp.float32),
        scratch_types=[
            pltpu.VMEM((NBUF, CHUNK), jnp.int32),
            pltpu.VMEM((NBUF, CHUNK, EMB), jnp.float32),
        ] + [pltpu.SemaphoreType.DMA] * (3 * NBUF),
        compiler_params=pltpu.CompilerParams(use_tc_tiling_on_sc=False),
    )
    def emb_kernel(x_hbm, table_hbm, out_hbm, idx_v, rows_v, *sems):
        sem_ld = sems[0:NBUF]
        sem_g = sems[NBUF:2 * NBUF]
        sem_st = sems[2 * NBUF:3 * NBUF]
        wid = lax.axis_index("s") * NUM_CORES + lax.axis_index("c")
        base = wid * B_PER_W

        def fire_ld(g, b):
            pltpu.async_copy(
                x_hbm.at[pl.ds(base + g * CHUNK, CHUNK)], idx_v.at[b],
                sem_ld[b])

        def wait_ld(b):
            pltpu.make_async_copy(
                x_hbm.at[pl.ds(base, CHUNK)], idx_v.at[b], sem_ld[b]).wait()

        def fire_gather(b):
            pltpu.async_copy(table_hbm.at[idx_v.at[b]], rows_v.at[b],
                             sem_g[b])

        def wait_gather(b):
            pltpu.make_async_copy(table_hbm.at[idx_v.at[b]], rows_v.at[b],
                                  sem_g[b]).wait()

        def fire_st(g, b):
            pltpu.async_copy(
                rows_v.at[b], out_hbm.at[pl.ds(base + g * CHUNK, CHUNK)],
                sem_st[b])

        def wait_st(b):
            pltpu.make_async_copy(
                rows_v.at[b], out_hbm.at[pl.ds(base, CHUNK)],
                sem_st[b]).wait()

        for b in range(NBUF):
            fire_ld(b, b)
        wait_ld(0)
        fire_gather(0)
        wait_ld(1)
        fire_gather(1)

        def body(g, b):
            wait_gather(b)
            fire_st(g, b)

            @pl.when(g + NBUF < N_CHUNKS)
            def _():
                fire_ld(g + NBUF, b)

            b2 = (b + 2) % NBUF

            @pl.when(g + 2 < N_CHUNKS)
            def _():
                wait_ld(b2)

                @pl.when(g >= 2)
                def _():
                    wait_st(b2)

                fire_gather(b2)

        @pl.loop(0, N_CHUNKS, step=NBUF)
        def _(h):
            for b in range(NBUF):
                body(h + b, b)

        for b in range(NBUF):
            wait_st(b)

    return emb_kernel


_emb_kernel = _make_kernel()


def kernel(x, table):
    flat = _emb_kernel(x.reshape(B_TOTAL), table)
    return flat.reshape(ROWS, COLS, EMB)


# speedup vs baseline: 3.8613x; 1.5025x over previous
import functools

import jax
import jax.numpy as jnp
from jax import lax
from jax.experimental import pallas as pl
from jax.experimental.pallas import tpu as pltpu
from jax.experimental.pallas import tpu_sc as plsc

VOCAB = 1_000_000
EMB = 16
ROWS = 16384
COLS = 200
B_TOTAL = ROWS * COLS

_info = plsc.get_sparse_core_info()
NUM_CORES = _info.num_cores
NUM_SUBCORES = _info.num_subcores
NW = NUM_CORES * NUM_SUBCORES

JT = COLS // 8
IB_TOTAL = ROWS // 128
IB_PER_W = IB_TOTAL // NW
NB = IB_PER_W * JT
BLK = 1024


def _make_kernel():
    mesh = plsc.VectorSubcoreMesh(core_axis_name="c", subcore_axis_name="s")

    @functools.partial(
        pl.kernel,
        mesh=mesh,
        out_type=jax.ShapeDtypeStruct((COLS, 2, 128, 8, 128), jnp.float32),
        scratch_types=[
            pltpu.VMEM((2, BLK), jnp.int32),
            pltpu.VMEM((2, BLK, EMB), jnp.float32),
            pltpu.VMEM((2, 8, EMB, 128), jnp.float32),
        ] + [pltpu.SemaphoreType.DMA] * 6,
        compiler_params=pltpu.CompilerParams(
            use_tc_tiling_on_sc=False, needs_layout_passes=False),
    )
    def emb_kernel(xt_hbm, table_hbm, out_hbm, idx_v, rows_v, rowst_v,
                   ld0, ld1, g0, g1, st0, st1):
        sem_ld = (ld0, ld1)
        sem_g = (g0, g1)
        sem_st = (st0, st1)
        wid = lax.axis_index("s") * NUM_CORES + lax.axis_index("c")
        iota = lax.iota(jnp.int32, 16)

        def block_coords(n):
            ib = n // JT
            jt = n - ib * JT
            return jt, wid * IB_PER_W + ib

        def fire_ld(n, b):
            jt, tcg = block_coords(n)
            pltpu.async_copy(xt_hbm.at[jt, tcg], idx_v.at[b], sem_ld[b])

        def wait_ld(b):
            pltpu.make_async_copy(xt_hbm.at[0, 0], idx_v.at[b],
                                  sem_ld[b]).wait()

        def fire_gather(b):
            pltpu.async_copy(table_hbm.at[idx_v.at[b]], rows_v.at[b],
                             sem_g[b])

        def wait_gather(b):
            pltpu.make_async_copy(table_hbm.at[idx_v.at[b]], rows_v.at[b],
                                  sem_g[b]).wait()

        def fire_st(n, b, j2, tr):
            jt, tcg = block_coords(n)
            pltpu.async_copy(
                rowst_v.at[b, j2, pl.ds(8 * tr, 8)],
                out_hbm.at[8 * jt + j2, tr, tcg], sem_st[b])

        def wait_st_all(b):
            for _ in range(16):
                pltpu.make_async_copy(
                    rowst_v.at[b, 0, pl.ds(0, 8)],
                    out_hbm.at[0, 0, 0], sem_st[b]).wait()

        def transpose_and_store(n, b):
            for j2 in range(8):
                @pl.loop(0, 128, step=4)
                def _(ii):
                    iota_l = lax.iota(jnp.int32, 16)
                    for u in range(4):
                        r = j2 * 128 + ii + u
                        row = plsc.load_gather(
                            rows_v.at[b], [jnp.full((16,), r, jnp.int32),
                                           iota_l])
                        plsc.store_scatter(
                            rowst_v.at[b, j2],
                            [iota_l, jnp.full((16,), ii + u, jnp.int32)],
                            row)

                jt, tcg = block_coords(n)
                for tr in range(2):
                    pltpu.sync_copy(
                        rowst_v.at[b, j2, pl.ds(8 * tr, 8)],
                        out_hbm.at[8 * jt + j2, tr, tcg])

        fire_ld(0, 0)
        fire_ld(1, 1)
        wait_ld(0)
        fire_gather(0)

        def body(n, b):
            wait_gather(b)

            @pl.when(n + 1 < NB)
            def _():
                wait_ld(1 - b)
                fire_gather(1 - b)

            @pl.when(n + 2 < NB)
            def _():
                fire_ld(n + 2, b)

            transpose_and_store(n, b)

        @pl.loop(0, NB, step=2)
        def _(h):
            for b in range(2):
                body(h + b, b)

    return emb_kernel


_emb_kernel = _make_kernel()


def kernel(x, table):
    xt = (x.transpose(1, 0).reshape(JT, 8, 128, 128)
          .transpose(0, 2, 1, 3).reshape(JT, 128, BLK))
    t = _emb_kernel(xt, table)
    return t.transpose((2, 4, 0, 1, 3)).reshape(ROWS, COLS, EMB)


# speedup vs baseline: 4.2193x; 1.0927x over previous
import functools

import jax
import jax.numpy as jnp
from jax import lax
from jax.experimental import pallas as pl
from jax.experimental.pallas import tpu as pltpu
from jax.experimental.pallas import tpu_sc as plsc

VOCAB = 1_000_000
EMB = 16
ROWS = 16384
COLS = 200
B_TOTAL = ROWS * COLS

_info = plsc.get_sparse_core_info()
NUM_CORES = _info.num_cores
NUM_SUBCORES = _info.num_subcores
NW = NUM_CORES * NUM_SUBCORES

JT = COLS // 8
IB_TOTAL = ROWS // 128
IB_PER_W = IB_TOTAL // NW
NB = IB_PER_W * JT
BLK = 1024


def _make_kernel():
    mesh = plsc.VectorSubcoreMesh(core_axis_name="c", subcore_axis_name="s")

    @functools.partial(
        pl.kernel,
        mesh=mesh,
        out_type=jax.ShapeDtypeStruct((COLS, 2, 128, 8, 128), jnp.float32),
        scratch_types=[
            pltpu.VMEM((2, BLK), jnp.int32),
            pltpu.VMEM((2, BLK, EMB), jnp.float32),
            pltpu.VMEM((2, 8, EMB, 128), jnp.float32),
        ] + [pltpu.SemaphoreType.DMA] * 6,
        compiler_params=pltpu.CompilerParams(
            use_tc_tiling_on_sc=False, needs_layout_passes=False),
    )
    def emb_kernel(xt_hbm, table_hbm, out_hbm, idx_v, rows_v, rowst_v,
                   ld0, ld1, g0, g1, st0, st1):
        sem_ld = (ld0, ld1)
        sem_g = (g0, g1)
        sem_st = (st0, st1)
        wid = lax.axis_index("s") * NUM_CORES + lax.axis_index("c")
        iota = lax.iota(jnp.int32, 16)

        def block_coords(n):
            ib = n // JT
            jt = n - ib * JT
            return jt, wid * IB_PER_W + ib

        def fire_ld(n, b):
            jt, tcg = block_coords(n)
            pltpu.async_copy(xt_hbm.at[jt, tcg], idx_v.at[b], sem_ld[b])

        def wait_ld(b):
            pltpu.make_async_copy(xt_hbm.at[0, 0], idx_v.at[b],
                                  sem_ld[b]).wait()

        def fire_gather(b):
            pltpu.async_copy(table_hbm.at[idx_v.at[b]], rows_v.at[b],
                             sem_g[b])

        def wait_gather(b):
            pltpu.make_async_copy(table_hbm.at[idx_v.at[b]], rows_v.at[b],
                                  sem_g[b]).wait()

        def fire_st(n, b, j2, tr):
            jt, tcg = block_coords(n)
            pltpu.async_copy(
                rowst_v.at[b, j2, pl.ds(8 * tr, 8)],
                out_hbm.at[8 * jt + j2, tr, tcg], sem_st[b])

        def wait_st_all(b):
            for _ in range(16):
                pltpu.make_async_copy(
                    rowst_v.at[b, 0, pl.ds(0, 8)],
                    out_hbm.at[0, 0, 0], sem_st[b]).wait()

        def transpose_and_store(n, b):
            for j2 in range(8):
                @pl.loop(0, 128, step=4)
                def _(ii):
                    iota_l = lax.iota(jnp.int32, 16)
                    for u in range(4):
                        r = j2 * 128 + ii + u
                        row = plsc.load_gather(
                            rows_v.at[b], [jnp.full((16,), r, jnp.int32),
                                           iota_l])
                        plsc.store_scatter(
                            rowst_v.at[b, j2],
                            [iota_l, jnp.full((16,), ii + u, jnp.int32)],
                            row)

                fire_st(n, b, j2, 0)
                fire_st(n, b, j2, 1)
            wait_st_all(b)

        fire_ld(0, 0)
        fire_ld(1, 1)
        wait_ld(0)
        fire_gather(0)

        def body(n, b):
            wait_gather(b)

            @pl.when(n + 1 < NB)
            def _():
                wait_ld(1 - b)
                fire_gather(1 - b)

            @pl.when(n + 2 < NB)
            def _():
                fire_ld(n + 2, b)

            transpose_and_store(n, b)

        @pl.loop(0, NB, step=2)
        def _(h):
            for b in range(2):
                body(h + b, b)

    return emb_kernel


_emb_kernel = _make_kernel()


def kernel(x, table):
    xt = (x.transpose(1, 0).reshape(JT, 8, 128, 128)
          .transpose(0, 2, 1, 3).reshape(JT, 128, BLK))
    t = _emb_kernel(xt, table)
    return t.transpose((2, 4, 0, 1, 3)).reshape(ROWS, COLS, EMB)
